# baseline (speedup 1.0000x reference)
import functools

import jax
import jax.numpy as jnp
from jax import lax
from jax.experimental import pallas as pl
from jax.experimental.pallas import tpu as pltpu
from jax.experimental.pallas import tpu_sc as plsc

_NPAD = 1 << 20
_NC = 2
_NS = 16
_NW = _NC * _NS
_PER_W = _NPAD // _NW
_BLK = 8192
_CH = 128
_G = _CH // 16
_DM1 = 255.0


@functools.partial(
    pl.kernel,
    out_type=jax.ShapeDtypeStruct((_NPAD,), jnp.float32),
    mesh=plsc.VectorSubcoreMesh(core_axis_name="c", subcore_axis_name="s"),
    scratch_types=[
        pltpu.VMEM((_BLK,), jnp.float32),
        pltpu.VMEM((_BLK,), jnp.float32),
        pltpu.VMEM((_BLK,), jnp.float32),
        pltpu.VMEM((8, _CH), jnp.int32),
        pltpu.VMEM((8, _CH), jnp.float32),
        pltpu.VMEM((3, _CH), jnp.float32),
        pltpu.VMEM((_BLK,), jnp.float32),
        pltpu.SemaphoreType.DMA,
    ],
)
def _trilinear(zs, ys, xs, grid, out, zb, yb, xb, idx_s, val_s, frac_s, ob,
               sem):
    wid = lax.axis_index("s") * _NC + lax.axis_index("c")
    base_w = wid * _PER_W

    for b in range(_PER_W // _BLK):
        base = pl.multiple_of(base_w + b * _BLK, _BLK)
        pltpu.sync_copy(zs.at[pl.ds(base, _BLK)], zb)
        pltpu.sync_copy(ys.at[pl.ds(base, _BLK)], yb)
        pltpu.sync_copy(xs.at[pl.ds(base, _BLK)], xb)

        def chunk(c, carry):
            co = c * _CH
            for g in range(_G):
                o = pl.multiple_of(co + g * 16, 16)
                s16 = pl.ds(o, 16)
                gs = pl.ds(g * 16, 16)
                z = jnp.minimum(jnp.maximum(zb[s16] * _DM1, 0.0), _DM1)
                y = jnp.minimum(jnp.maximum(yb[s16] * _DM1, 0.0), _DM1)
                x = jnp.minimum(jnp.maximum(xb[s16] * _DM1, 0.0), _DM1)
                zi = z.astype(jnp.int32)
                yi = y.astype(jnp.int32)
                xi = x.astype(jnp.int32)
                frac_s[0, gs] = z - zi.astype(jnp.float32)
                frac_s[1, gs] = y - yi.astype(jnp.float32)
                frac_s[2, gs] = x - xi.astype(jnp.float32)
                z1 = jnp.minimum(zi + 1, 255)
                y1 = jnp.minimum(yi + 1, 255)
                x1 = jnp.minimum(xi + 1, 255)
                zo0 = zi * 65536
                zo1 = z1 * 65536
                yo0 = yi * 256
                yo1 = y1 * 256
                b00 = zo0 + yo0
                b01 = zo0 + yo1
                b10 = zo1 + yo0
                b11 = zo1 + yo1
                idx_s[0, gs] = b00 + xi
                idx_s[1, gs] = b00 + x1
                idx_s[2, gs] = b01 + xi
                idx_s[3, gs] = b01 + x1
                idx_s[4, gs] = b10 + xi
                idx_s[5, gs] = b10 + x1
                idx_s[6, gs] = b11 + xi
                idx_s[7, gs] = b11 + x1
            copies = [
                pltpu.async_copy(grid.at[idx_s.at[k]], val_s.at[k], sem)
                for k in range(8)
            ]
            for cp in copies:
                cp.wait()
            for g in range(_G):
                gs = pl.ds(g * 16, 16)
                fz = frac_s[0, gs]
                fy = frac_s[1, gs]
                fx = frac_s[2, gs]
                c000 = val_s[0, gs]
                c001 = val_s[1, gs]
                c010 = val_s[2, gs]
                c011 = val_s[3, gs]
                c100 = val_s[4, gs]
                c101 = val_s[5, gs]
                c110 = val_s[6, gs]
                c111 = val_s[7, gs]
                c00 = c000 + fx * (c001 - c000)
                c01 = c010 + fx * (c011 - c010)
                c10 = c100 + fx * (c101 - c100)
                c11 = c110 + fx * (c111 - c110)
                c0 = c00 + fy * (c01 - c00)
                c1 = c10 + fy * (c11 - c10)
                o = pl.multiple_of(co + g * 16, 16)
                ob[pl.ds(o, 16)] = c0 + fz * (c1 - c0)
            return carry

        lax.fori_loop(0, _BLK // _CH, chunk, 0)
        pltpu.sync_copy(ob, out.at[pl.ds(base, _BLK)])


def kernel(voxel_grid, coordinates):
    n = coordinates.shape[0]
    c = voxel_grid.shape[0]
    coords = jnp.pad(coordinates, ((0, _NPAD - n), (0, 0))).T
    zs = coords[0] + 0.0
    ys = coords[1] + 0.0
    xs = coords[2] + 0.0
    grid = voxel_grid.reshape(-1)
    occ = _trilinear(zs, ys, xs, grid)
    return occ[:n].reshape(c, n)
